# Initial kernel scaffold; baseline (speedup 1.0000x reference)
import jax
import jax.numpy as jnp
from jax.experimental import pallas as pl


def kernel(weight, inputs):
    raise NotImplementedError("write your pallas kernel here")


# speedup vs baseline: 2.7584x; 2.7584x over previous
import functools

import jax
import jax.numpy as jnp
from jax import lax
from jax.experimental import pallas as pl
from jax.experimental.pallas import tpu as pltpu
from jax.experimental.pallas import tpu_sc as plsc

D = 128
NC = 2
NS = 16
NW = NC * NS
CHUNK = 128


@functools.lru_cache(maxsize=None)
def _make_gather(B: int):
    assert B % (NW * CHUNK) == 0
    bpw = B // NW
    nch = bpw // CHUNK

    mesh = plsc.VectorSubcoreMesh(core_axis_name="c", subcore_axis_name="s")

    @functools.partial(
        pl.kernel,
        mesh=mesh,
        out_type=jax.ShapeDtypeStruct((B, D), jnp.float32),
        scratch_types=[
            pltpu.VMEM((CHUNK,), jnp.int32),
            pltpu.VMEM((CHUNK, D), jnp.float32),
            pltpu.SemaphoreType.DMA,
        ],
    )
    def gather_kernel(table_hbm, idx_hbm, out_hbm, idx_v, rows_v, sem):
        wid = lax.axis_index("s") * NC + lax.axis_index("c")
        base = wid * bpw

        def body(i, carry):
            off = base + i * CHUNK
            pltpu.sync_copy(idx_hbm.at[pl.ds(off, CHUNK)], idx_v)
            pltpu.async_copy(table_hbm.at[idx_v], rows_v, sem).wait()
            pltpu.sync_copy(rows_v, out_hbm.at[pl.ds(off, CHUNK)])
            return carry

        lax.fori_loop(0, nch, body, 0)

    return gather_kernel


def kernel(weight, inputs):
    b0, b1 = inputs.shape
    idx = inputs.reshape(b0 * b1).astype(jnp.int32)
    out = _make_gather(b0 * b1)(weight, idx)
    return out.reshape(b0, b1, D)


# speedup vs baseline: 3.3389x; 1.2104x over previous
import functools

import jax
import jax.numpy as jnp
from jax import lax
from jax.experimental import pallas as pl
from jax.experimental.pallas import tpu as pltpu
from jax.experimental.pallas import tpu_sc as plsc

D = 128
NC = 2
NS = 16
NW = NC * NS
CHUNK = 128
NBUF = 5
LAG = 2


@functools.lru_cache(maxsize=None)
def _make_gather(B: int):
    assert B % (NW * CHUNK) == 0
    bpw = B // NW
    nch = bpw // CHUNK
    assert nch > NBUF >= LAG + 1 and (nch - NBUF) % NBUF == 0

    mesh = plsc.VectorSubcoreMesh(core_axis_name="c", subcore_axis_name="s")

    @functools.partial(
        pl.kernel,
        mesh=mesh,
        out_type=jax.ShapeDtypeStruct((B, D), jnp.float32),
        scratch_types=[
            pltpu.VMEM((bpw,), jnp.int32),
            pltpu.VMEM((NBUF, CHUNK, D), jnp.float32),
        ]
        + [pltpu.SemaphoreType.DMA] * NBUF
        + [pltpu.SemaphoreType.DMA] * NBUF,
    )
    def gather_kernel(table_hbm, idx_hbm, out_hbm, idx_v, rows_v, *sems):
        gsem = sems[:NBUF]
        osem = sems[NBUF:]
        wid = lax.axis_index("s") * NC + lax.axis_index("c")
        base = wid * bpw

        pltpu.sync_copy(idx_hbm.at[pl.ds(base, bpw)], idx_v)

        def start_gather(i, b):
            pltpu.async_copy(
                table_hbm.at[idx_v.at[pl.ds(i * CHUNK, CHUNK)]], rows_v.at[b], gsem[b]
            )

        def start_out(i, b):
            pltpu.async_copy(
                rows_v.at[b], out_hbm.at[pl.ds(base + i * CHUNK, CHUNK)], osem[b]
            )

        def wait_gather(b):
            pltpu.make_async_copy(
                table_hbm.at[pl.ds(0, CHUNK)], rows_v.at[b], gsem[b]
            ).wait()

        def wait_out(b):
            pltpu.make_async_copy(
                rows_v.at[b], out_hbm.at[pl.ds(base, CHUNK)], osem[b]
            ).wait()

        for i in range(NBUF):
            start_gather(i, i)
            if i >= LAG:
                wait_gather(i - LAG)
                start_out(i - LAG, i - LAG)

        def outer(j0, carry):
            for u in range(NBUF):
                i = j0 + u
                b = u
                b2 = (u - LAG) % NBUF
                wait_out(b)
                start_gather(i, b)
                wait_gather(b2)
                start_out(i - LAG, b2)
            return carry

        lax.fori_loop(0, (nch - NBUF) // NBUF, lambda t, c: outer(NBUF + t * NBUF, c),
                      0, unroll=False)

        for i in range(nch - LAG, nch):
            b = i % NBUF
            wait_gather(b)
            start_out(i, b)
        for b in range(NBUF):
            wait_out(b)

    return gather_kernel


def kernel(weight, inputs):
    b0, b1 = inputs.shape
    B = b0 * b1
    idx = inputs.reshape(B).astype(jnp.int32)
    out = _make_gather(B)(weight, idx)
    return out.reshape(b0, b1, D)


# speedup vs baseline: 3.3484x; 1.0029x over previous
import functools

import jax
import jax.numpy as jnp
from jax import lax
from jax.experimental import pallas as pl
from jax.experimental.pallas import tpu as pltpu
from jax.experimental.pallas import tpu_sc as plsc

D = 128
NC = 2
NS = 16
NW = NC * NS
CHUNK = 128
NBUF = 5
LAG = 3


@functools.lru_cache(maxsize=None)
def _make_gather(B: int):
    assert B % (NW * CHUNK) == 0
    bpw = B // NW
    nch = bpw // CHUNK
    assert nch > NBUF >= LAG + 1 and (nch - NBUF) % NBUF == 0

    mesh = plsc.VectorSubcoreMesh(core_axis_name="c", subcore_axis_name="s")

    @functools.partial(
        pl.kernel,
        mesh=mesh,
        out_type=jax.ShapeDtypeStruct((B, D), jnp.float32),
        scratch_types=[
            pltpu.VMEM((bpw,), jnp.int32),
            pltpu.VMEM((NBUF, CHUNK, D), jnp.float32),
        ]
        + [pltpu.SemaphoreType.DMA] * NBUF
        + [pltpu.SemaphoreType.DMA] * NBUF,
    )
    def gather_kernel(table_hbm, idx_hbm, out_hbm, idx_v, rows_v, *sems):
        gsem = sems[:NBUF]
        osem = sems[NBUF:]
        wid = lax.axis_index("s") * NC + lax.axis_index("c")
        base = wid * bpw

        pltpu.sync_copy(idx_hbm.at[pl.ds(base, bpw)], idx_v)

        def start_gather(i, b):
            pltpu.async_copy(
                table_hbm.at[idx_v.at[pl.ds(i * CHUNK, CHUNK)]], rows_v.at[b], gsem[b]
            )

        def start_out(i, b):
            pltpu.async_copy(
                rows_v.at[b], out_hbm.at[pl.ds(base + i * CHUNK, CHUNK)], osem[b]
            )

        def wait_gather(b):
            pltpu.make_async_copy(
                table_hbm.at[pl.ds(0, CHUNK)], rows_v.at[b], gsem[b]
            ).wait()

        def wait_out(b):
            pltpu.make_async_copy(
                rows_v.at[b], out_hbm.at[pl.ds(base, CHUNK)], osem[b]
            ).wait()

        for i in range(NBUF):
            start_gather(i, i)
            if i >= LAG:
                wait_gather(i - LAG)
                start_out(i - LAG, i - LAG)

        def outer(j0, carry):
            for u in range(NBUF):
                i = j0 + u
                b = u
                b2 = (u - LAG) % NBUF
                wait_out(b)
                start_gather(i, b)
                wait_gather(b2)
                start_out(i - LAG, b2)
            return carry

        lax.fori_loop(0, (nch - NBUF) // NBUF, lambda t, c: outer(NBUF + t * NBUF, c),
                      0, unroll=False)

        for i in range(nch - LAG, nch):
            b = i % NBUF
            wait_gather(b)
            start_out(i, b)
        for b in range(NBUF):
            wait_out(b)

    return gather_kernel


def kernel(weight, inputs):
    b0, b1 = inputs.shape
    B = b0 * b1
    idx = inputs.reshape(B).astype(jnp.int32)
    out = _make_gather(B)(weight, idx)
    return out.reshape(b0, b1, D)


# speedup vs baseline: 5.9576x; 1.7792x over previous
import functools

import jax
import jax.numpy as jnp
from jax import lax
from jax.experimental import pallas as pl
from jax.experimental.pallas import tpu as pltpu
from jax.experimental.pallas import tpu_sc as plsc

D = 128
NC = 2
NS = 16
NW = NC * NS
SEQ_CHUNK = 8
NBUF = 2
LAG = 1


@functools.lru_cache(maxsize=None)
def _make_gather(S: int, T: int):
    assert S % (NW * SEQ_CHUNK) == 0
    spw = S // NW
    nch = spw // SEQ_CHUNK
    assert nch > NBUF >= LAG + 1 and (nch - NBUF) % NBUF == 0

    mesh = plsc.VectorSubcoreMesh(core_axis_name="c", subcore_axis_name="s")

    @functools.partial(
        pl.kernel,
        mesh=mesh,
        out_type=jax.ShapeDtypeStruct((S, T, D), jnp.float32),
        scratch_types=[
            pltpu.VMEM((spw, T), jnp.int32),
            pltpu.VMEM((NBUF, SEQ_CHUNK, T, D), jnp.float32),
        ]
        + [pltpu.SemaphoreType.DMA] * NBUF
        + [pltpu.SemaphoreType.DMA] * NBUF,
    )
    def gather_kernel(table_hbm, idx_hbm, out_hbm, idx_v, rows_v, *sems):
        gsem = sems[:NBUF]
        osem = sems[NBUF:]
        wid = lax.axis_index("s") * NC + lax.axis_index("c")
        base = wid * spw

        pltpu.sync_copy(idx_hbm.at[pl.ds(base, spw)], idx_v)

        def start_gather(i, b):
            for k in range(SEQ_CHUNK):
                pltpu.async_copy(
                    table_hbm.at[idx_v.at[i * SEQ_CHUNK + k]],
                    rows_v.at[b].at[k],
                    gsem[b],
                )

        def start_out(i, b):
            pltpu.async_copy(
                rows_v.at[b],
                out_hbm.at[pl.ds(base + i * SEQ_CHUNK, SEQ_CHUNK)],
                osem[b],
            )

        def wait_gather(b):
            pltpu.make_async_copy(
                out_hbm.at[pl.ds(base, SEQ_CHUNK)], rows_v.at[b], gsem[b]
            ).wait()

        def wait_out(b):
            pltpu.make_async_copy(
                rows_v.at[b], out_hbm.at[pl.ds(base, SEQ_CHUNK)], osem[b]
            ).wait()

        for i in range(NBUF):
            start_gather(i, i)
            if i >= LAG:
                wait_gather(i - LAG)
                start_out(i - LAG, i - LAG)

        def outer(j0, carry):
            for u in range(NBUF):
                i = j0 + u
                b = u
                b2 = (u - LAG) % NBUF
                wait_out(b)
                start_gather(i, b)
                wait_gather(b2)
                start_out(i - LAG, b2)
            return carry

        lax.fori_loop(0, (nch - NBUF) // NBUF, lambda t, c: outer(NBUF + t * NBUF, c),
                      0, unroll=False)

        for i in range(nch - LAG, nch):
            b = i % NBUF
            wait_gather(b)
            start_out(i, b)
        for b in range(NBUF):
            wait_out(b)

    return gather_kernel


def kernel(weight, inputs):
    S, T = inputs.shape
    return _make_gather(S, T)(weight, inputs.astype(jnp.int32))


# speedup vs baseline: 10.6851x; 1.7935x over previous
import functools

import jax
import jax.numpy as jnp
from jax import lax
from jax.experimental import pallas as pl
from jax.experimental.pallas import tpu as pltpu
from jax.experimental.pallas import tpu_sc as plsc

D = 128
NC = 2
NS = 16
NW = NC * NS
CB = 128
NBUF = 5
LAG = 2


@functools.lru_cache(maxsize=None)
def _make_gather(S: int, T: int):
    assert S % (NW * CB) == 0
    nch = T
    assert nch > NBUF >= LAG + 1 and (nch - NBUF) % NBUF == 0

    mesh = plsc.VectorSubcoreMesh(core_axis_name="c", subcore_axis_name="s")

    @functools.partial(
        pl.kernel,
        mesh=mesh,
        out_type=jax.ShapeDtypeStruct((T, S, D), jnp.float32),
        scratch_types=[
            pltpu.VMEM((T, CB), jnp.int32),
            pltpu.VMEM((NBUF, CB, D), jnp.float32),
        ]
        + [pltpu.SemaphoreType.DMA] * NBUF
        + [pltpu.SemaphoreType.DMA] * NBUF,
    )
    def gather_kernel(table_hbm, idx_hbm, out_hbm, idx_v, rows_v, *sems):
        gsem = sems[:NBUF]
        osem = sems[NBUF:]
        wid = lax.axis_index("s") * NC + lax.axis_index("c")
        s0 = wid * CB

        pltpu.sync_copy(idx_hbm.at[:, pl.ds(s0, CB)], idx_v)

        def start_gather(t, b):
            pltpu.async_copy(table_hbm.at[idx_v.at[t]], rows_v.at[b], gsem[b])

        def start_out(t, b):
            pltpu.async_copy(
                rows_v.at[b], out_hbm.at[t].at[pl.ds(s0, CB)], osem[b]
            )

        def wait_gather(b):
            pltpu.make_async_copy(
                table_hbm.at[pl.ds(0, CB)], rows_v.at[b], gsem[b]
            ).wait()

        def wait_out(b):
            pltpu.make_async_copy(
                rows_v.at[b], out_hbm.at[0].at[pl.ds(s0, CB)], osem[b]
            ).wait()

        for t in range(NBUF):
            start_gather(t, t)
            if t >= LAG:
                wait_gather(t - LAG)
                start_out(t - LAG, t - LAG)

        def outer(j0, carry):
            for u in range(NBUF):
                t = j0 + u
                b = u
                b2 = (u - LAG) % NBUF
                wait_out(b)
                start_gather(t, b)
                wait_gather(b2)
                start_out(t - LAG, b2)
            return carry

        lax.fori_loop(0, (nch - NBUF) // NBUF, lambda i, c: outer(NBUF + i * NBUF, c),
                      0, unroll=False)

        for t in range(nch - LAG, nch):
            b = t % NBUF
            wait_gather(b)
            start_out(t, b)
        for b in range(NBUF):
            wait_out(b)

    return gather_kernel


def kernel(weight, inputs):
    S, T = inputs.shape
    idx_t = jnp.transpose(inputs).astype(jnp.int32)
    out_t = _make_gather(S, T)(weight, idx_t)
    return jnp.transpose(out_t, (1, 0, 2))
